# baseline (speedup 1.0000x reference)
import jax
import jax.numpy as jnp
from jax.experimental import pallas as pl
from jax.experimental.pallas import tpu as pltpu

_E = 16
_K = 2
_H = 1024
_F = 2048
_CAP = 24
_EPS = 1e-06
_N = 64
_S = _E * _CAP

_FB = 2048
_NF = _F // _FB


def _moe_kernel(x_ref, rms_ref, wr_ref, bias_ref, gate_ref, up_ref, down_ref,
                out_ref, buf_scr, g_scr, y_scr):
    e = pl.program_id(0)
    f = pl.program_id(1)

    @pl.when(jnp.logical_and(e == 0, f == 0))
    def _prologue():
        x = x_ref[...]
        v = jnp.mean(x * x, axis=-1, keepdims=True)
        h = rms_ref[...] * (x * jax.lax.rsqrt(v + _EPS))
        logits = jnp.dot(h, wr_ref[...], preferred_element_type=jnp.float32)
        scores = jax.nn.sigmoid(logits)
        choice = scores + bias_ref[...]
        ei = jax.lax.broadcasted_iota(jnp.int32, (_N, _E), 1)
        m1 = jnp.max(choice, axis=1, keepdims=True)
        idx1 = jnp.min(jnp.where(choice == m1, ei, _E), axis=1, keepdims=True)
        choice2 = jnp.where(ei == idx1, -jnp.inf, choice)
        m2 = jnp.max(choice2, axis=1, keepdims=True)
        idx2 = jnp.min(jnp.where(choice2 == m2, ei, _E), axis=1, keepdims=True)
        w1 = jnp.sum(jnp.where(ei == idx1, scores, 0.0), axis=1, keepdims=True)
        w2 = jnp.sum(jnp.where(ei == idx2, scores, 0.0), axis=1, keepdims=True)
        ws = w1 + w2 + 1e-09
        w1 = w1 / ws
        w2 = w2 / ws
        oh = ((ei == idx1).astype(jnp.float32)
              + (ei == idx2).astype(jnp.float32))
        ti = jax.lax.broadcasted_iota(jnp.int32, (_N, _N), 0)
        tj = jax.lax.broadcasted_iota(jnp.int32, (_N, _N), 1)
        ltri = (tj < ti).astype(jnp.float32)
        cnt_before = jnp.dot(ltri, oh, preferred_element_type=jnp.float32)
        pos1 = jnp.sum(jnp.where(ei == idx1, cnt_before, 0.0),
                       axis=1, keepdims=True).astype(jnp.int32)
        pos2 = jnp.sum(jnp.where(ei == idx2, cnt_before, 0.0),
                       axis=1, keepdims=True).astype(jnp.int32)
        slot1 = idx1 * _CAP + pos1
        slot2 = idx2 * _CAP + pos2
        ok1 = pos1 < _CAP
        ok2 = pos2 < _CAP
        sj = jax.lax.broadcasted_iota(jnp.int32, (_N, _S), 1)
        g_scr[...] = (jnp.where((sj == slot1) & ok1, w1, 0.0)
                      + jnp.where((sj == slot2) & ok2, w2, 0.0))
        ind = (jnp.where((sj == slot1) & ok1, 1.0, 0.0)
               + jnp.where((sj == slot2) & ok2, 1.0, 0.0))
        buf_scr[...] = jax.lax.dot_general(
            ind, h, (((0,), (0,)), ((), ())),
            preferred_element_type=jnp.float32)

    be = buf_scr[pl.ds(e * _CAP, _CAP), :]
    g = jnp.dot(be, gate_ref[0], preferred_element_type=jnp.float32)
    u = jnp.dot(be, up_ref[0], preferred_element_type=jnp.float32)
    act = g * jax.nn.sigmoid(g) * u
    contrib = jnp.dot(act, down_ref[0], preferred_element_type=jnp.float32)

    @pl.when(f == 0)
    def _init():
        y_scr[pl.ds(e * _CAP, _CAP), :] = contrib

    @pl.when(f > 0)
    def _acc():
        y_scr[pl.ds(e * _CAP, _CAP), :] += contrib

    @pl.when(jnp.logical_and(e == _E - 1, f == _NF - 1))
    def _epilogue():
        out_ref[...] = x_ref[...] + jnp.dot(
            g_scr[...], y_scr[...], preferred_element_type=jnp.float32)


def kernel(x, rms_w, W_router, bias_corr, W_gate, W_up, W_down):
    return pl.pallas_call(
        _moe_kernel,
        grid=(_E, _NF),
        in_specs=[
            pl.BlockSpec((_N, _H), lambda e, f: (0, 0)),
            pl.BlockSpec((1, _H), lambda e, f: (0, 0)),
            pl.BlockSpec((_H, _E), lambda e, f: (0, 0)),
            pl.BlockSpec((1, _E), lambda e, f: (0, 0)),
            pl.BlockSpec((1, _H, _FB), lambda e, f: (e, 0, f)),
            pl.BlockSpec((1, _H, _FB), lambda e, f: (e, 0, f)),
            pl.BlockSpec((1, _FB, _H), lambda e, f: (e, f, 0)),
        ],
        out_specs=pl.BlockSpec((_N, _H), lambda e, f: (0, 0)),
        out_shape=jax.ShapeDtypeStruct((_N, _H), jnp.float32),
        scratch_shapes=[
            pltpu.VMEM((_S, _H), jnp.float32),
            pltpu.VMEM((_N, _S), jnp.float32),
            pltpu.VMEM((_S, _H), jnp.float32),
        ],
        compiler_params=pltpu.CompilerParams(
            dimension_semantics=("arbitrary", "arbitrary")),
    )(x, rms_w.reshape(1, _H), W_router, bias_corr.reshape(1, _E),
      W_gate, W_up, W_down)


# speedup vs baseline: 1.0283x; 1.0283x over previous
import jax
import jax.numpy as jnp
from jax.experimental import pallas as pl
from jax.experimental.pallas import tpu as pltpu

_E = 16
_K = 2
_H = 1024
_F = 2048
_CAP = 24
_EPS = 1e-06
_N = 64
_S = _E * _CAP

_FB = 1024
_NF = _F // _FB


def _moe_kernel(x_ref, rms_ref, wr_ref, bias_ref, gate_ref, up_ref, down_ref,
                out_ref, buf_scr, g_scr, y_scr):
    e = pl.program_id(0)
    f = pl.program_id(1)

    @pl.when(jnp.logical_and(e == 0, f == 0))
    def _prologue():
        x = x_ref[...]
        v = jnp.mean(x * x, axis=-1, keepdims=True)
        h = rms_ref[...] * (x * jax.lax.rsqrt(v + _EPS))
        logits = jnp.dot(h, wr_ref[...], preferred_element_type=jnp.float32)
        scores = jax.nn.sigmoid(logits)
        choice = scores + bias_ref[...]
        ei = jax.lax.broadcasted_iota(jnp.int32, (_N, _E), 1)
        m1 = jnp.max(choice, axis=1, keepdims=True)
        idx1 = jnp.min(jnp.where(choice == m1, ei, _E), axis=1, keepdims=True)
        choice2 = jnp.where(ei == idx1, -jnp.inf, choice)
        m2 = jnp.max(choice2, axis=1, keepdims=True)
        idx2 = jnp.min(jnp.where(choice2 == m2, ei, _E), axis=1, keepdims=True)
        w1 = jnp.sum(jnp.where(ei == idx1, scores, 0.0), axis=1, keepdims=True)
        w2 = jnp.sum(jnp.where(ei == idx2, scores, 0.0), axis=1, keepdims=True)
        ws = w1 + w2 + 1e-09
        w1 = w1 / ws
        w2 = w2 / ws
        oh = ((ei == idx1).astype(jnp.float32)
              + (ei == idx2).astype(jnp.float32))
        ti = jax.lax.broadcasted_iota(jnp.int32, (_N, _N), 0)
        tj = jax.lax.broadcasted_iota(jnp.int32, (_N, _N), 1)
        ltri = (tj < ti).astype(jnp.float32)
        cnt_before = jnp.dot(ltri, oh, preferred_element_type=jnp.float32)
        pos1 = jnp.sum(jnp.where(ei == idx1, cnt_before, 0.0),
                       axis=1, keepdims=True).astype(jnp.int32)
        pos2 = jnp.sum(jnp.where(ei == idx2, cnt_before, 0.0),
                       axis=1, keepdims=True).astype(jnp.int32)
        slot1 = idx1 * _CAP + pos1
        slot2 = idx2 * _CAP + pos2
        ok1 = pos1 < _CAP
        ok2 = pos2 < _CAP
        sj = jax.lax.broadcasted_iota(jnp.int32, (_N, _S), 1)
        g_scr[...] = (jnp.where((sj == slot1) & ok1, w1, 0.0)
                      + jnp.where((sj == slot2) & ok2, w2, 0.0))
        ind = (jnp.where((sj == slot1) & ok1, 1.0, 0.0)
               + jnp.where((sj == slot2) & ok2, 1.0, 0.0))
        buf_scr[...] = jax.lax.dot_general(
            ind, h, (((0,), (0,)), ((), ())),
            preferred_element_type=jnp.float32)

    be = buf_scr[pl.ds(e * _CAP, _CAP), :]
    g = jnp.dot(be, gate_ref[0], preferred_element_type=jnp.float32)
    u = jnp.dot(be, up_ref[0], preferred_element_type=jnp.float32)
    act = g * jax.nn.sigmoid(g) * u
    contrib = jnp.dot(act, down_ref[0], preferred_element_type=jnp.float32)

    @pl.when(f == 0)
    def _init():
        y_scr[pl.ds(e * _CAP, _CAP), :] = contrib

    @pl.when(f > 0)
    def _acc():
        y_scr[pl.ds(e * _CAP, _CAP), :] += contrib

    @pl.when(jnp.logical_and(e == _E - 1, f == _NF - 1))
    def _epilogue():
        out_ref[...] = x_ref[...] + jnp.dot(
            g_scr[...], y_scr[...], preferred_element_type=jnp.float32)


def kernel(x, rms_w, W_router, bias_corr, W_gate, W_up, W_down):
    return pl.pallas_call(
        _moe_kernel,
        grid=(_E, _NF),
        in_specs=[
            pl.BlockSpec((_N, _H), lambda e, f: (0, 0)),
            pl.BlockSpec((1, _H), lambda e, f: (0, 0)),
            pl.BlockSpec((_H, _E), lambda e, f: (0, 0)),
            pl.BlockSpec((1, _E), lambda e, f: (0, 0)),
            pl.BlockSpec((1, _H, _FB), lambda e, f: (e, 0, f)),
            pl.BlockSpec((1, _H, _FB), lambda e, f: (e, 0, f)),
            pl.BlockSpec((1, _FB, _H), lambda e, f: (e, f, 0)),
        ],
        out_specs=pl.BlockSpec((_N, _H), lambda e, f: (0, 0)),
        out_shape=jax.ShapeDtypeStruct((_N, _H), jnp.float32),
        scratch_shapes=[
            pltpu.VMEM((_S, _H), jnp.float32),
            pltpu.VMEM((_N, _S), jnp.float32),
            pltpu.VMEM((_S, _H), jnp.float32),
        ],
        compiler_params=pltpu.CompilerParams(
            dimension_semantics=("arbitrary", "arbitrary")),
    )(x, rms_w.reshape(1, _H), W_router, bias_corr.reshape(1, _E),
      W_gate, W_up, W_down)
